# Initial kernel scaffold; baseline (speedup 1.0000x reference)
import jax
import jax.numpy as jnp
from jax.experimental import pallas as pl


def kernel(x, W1, g1, b1, W2, g2, b2, W3, g3, b3, W4, g4, b4, W5, g5, b5, Wmu, bmu, Wlv, blv, Wd1, bd1, Wd2, bd2, Wp, bp, eps):
    raise NotImplementedError("write your pallas kernel here")


# speedup vs baseline: 6.3735x; 6.3735x over previous
import functools

import jax
import jax.numpy as jnp
from jax import lax
from jax.experimental import pallas as pl
from jax.experimental.pallas import tpu as pltpu
from jax.experimental.pallas import tpu_sc as plsc

KNN = 20
CP = 128
_NEG = -3.4e38


def _dot(a, b):
    return lax.dot_general(a, b, (((1,), (1,)), ((), ())),
                           preferred_element_type=jnp.float32)


def _twosum(s, v):
    t = s + v
    z = t - s
    e = (s - (t - z)) + (v - z)
    return t, e


def _comp_colsum(y):
    M, C = y.shape
    s = y[0:8]
    c = jnp.zeros((8, C), jnp.float32)
    for i in range(1, M // 8):
        s, e = _twosum(s, y[i * 8:(i + 1) * 8])
        c = c + e
    hi, lo = s[0:1], c[0:1]
    for r in range(1, 8):
        hi, e = _twosum(hi, s[r:r + 1])
        lo = lo + (e + c[r:r + 1])
    return hi, lo


def _comp_combine(rows_hi, rows_lo):
    NP = rows_hi.shape[0]
    hi, lo = rows_hi[0:1], rows_lo[0:1]
    for r in range(1, NP):
        hi, e = _twosum(hi, rows_hi[r:r + 1])
        lo = lo + (e + rows_lo[r:r + 1])
    return hi + lo


@functools.lru_cache(maxsize=None)
def _build_edge(B, N, R):
    def kern(xb_ref, xa_ref, idx_ref):
        b = pl.program_id(0)
        xb = xb_ref[0]
        xa = xa_ref[0]
        xxr = jnp.transpose(jnp.sum(xa * xa, axis=1, keepdims=True))
        xxc = jnp.sum(xb * xb, axis=1, keepdims=True)
        score = (2.0 * _dot(xb, xa) - xxc) - xxr
        iota = lax.broadcasted_iota(jnp.int32, (R, N), 1)
        cols = []
        for _ in range(KNN):
            m = jnp.max(score, axis=1, keepdims=True)
            cand = jnp.where(score == m, iota, N)
            sel = jnp.min(cand, axis=1, keepdims=True)
            cols.append(sel)
            score = jnp.where(iota == sel, _NEG, score)
        idx_ref[0] = jnp.concatenate(cols, axis=1) + b * N

    return pl.pallas_call(
        kern,
        grid=(B, N // R),
        in_specs=[
            pl.BlockSpec((1, R, CP), lambda b, i: (b, i, 0)),
            pl.BlockSpec((1, N, CP), lambda b, i: (b, 0, 0)),
        ],
        out_specs=pl.BlockSpec((1, R, KNN), lambda b, i: (b, i, 0)),
        out_shape=jax.ShapeDtypeStruct((B, N, KNN), jnp.int32),
    )


@functools.lru_cache(maxsize=None)
def _build_gather(NROWS):
    NC = 2
    NW = 32
    RPW = NROWS // NW
    P = 4
    CH = P * KNN
    NCHUNK = RPW // P
    mesh = plsc.VectorSubcoreMesh(core_axis_name="c", subcore_axis_name="s")

    @functools.partial(
        pl.kernel, mesh=mesh,
        out_type=jax.ShapeDtypeStruct((NROWS * KNN, CP), jnp.float32),
        scratch_types=[
            pltpu.VMEM((RPW * KNN,), jnp.int32),
            pltpu.VMEM((CH, CP), jnp.float32),
            pltpu.VMEM((CH, CP), jnp.float32),
            pltpu.SemaphoreType.DMA,
            pltpu.SemaphoreType.DMA,
        ],
    )
    def kern(x_hbm, idx_hbm, xg_hbm, idx_v, rows0, rows1, sem0, sem1):
        wid = lax.axis_index("s") * NC + lax.axis_index("c")
        base = wid * RPW
        pltpu.sync_copy(idx_hbm.at[pl.ds(base * KNN, RPW * KNN)], idx_v)
        obase = base * KNN
        pltpu.async_copy(x_hbm.at[idx_v.at[pl.ds(0, CH)]], rows0, sem0)

        def body(ci2, carry):
            c0 = ci2 * 2
            pltpu.make_async_copy(x_hbm.at[idx_v.at[pl.ds(0, CH)]],
                                  rows0, sem0).wait()
            pltpu.async_copy(
                x_hbm.at[idx_v.at[pl.ds((c0 + 1) * CH, CH)]], rows1, sem1)
            pltpu.sync_copy(rows0, xg_hbm.at[pl.ds(obase + c0 * CH, CH)])

            @pl.when(c0 + 2 < NCHUNK)
            def _():
                pltpu.async_copy(
                    x_hbm.at[idx_v.at[pl.ds((c0 + 2) * CH, CH)]], rows0, sem0)

            pltpu.make_async_copy(x_hbm.at[idx_v.at[pl.ds(0, CH)]],
                                  rows1, sem1).wait()
            pltpu.sync_copy(rows1, xg_hbm.at[pl.ds(obase + (c0 + 1) * CH, CH)])
            return carry

        lax.fori_loop(0, NCHUNK // 2, body, 0)

    return kern


@functools.lru_cache(maxsize=None)
def _build_conv(B, N, R, Cin, Cout):
    NB = N // R

    def kern(xg_ref, x_ref, w_ref, m_ref, s_ref):
        xg = xg_ref[0][:, :Cin]
        ctr = x_ref[0][:, :Cin]
        ctr_rep = jnp.broadcast_to(
            ctr[:, None, :], (R, KNN, Cin)).reshape(R * KNN, Cin)
        feat = jnp.concatenate([xg - ctr_rep, ctr_rep], axis=1)
        y = _dot(feat, w_ref[...])
        y3 = y.reshape(R, KNN, Cout)
        m_ref[0] = jnp.max(y3, axis=1)
        s_hi, s_lo = _comp_colsum(y)
        s_ref[0] = jnp.concatenate([s_hi, s_lo], axis=0)

    return pl.pallas_call(
        kern,
        grid=(B, NB),
        in_specs=[
            pl.BlockSpec((1, R * KNN, CP), lambda b, i: (b * NB + i, 0, 0)),
            pl.BlockSpec((1, R, CP), lambda b, i: (b, i, 0)),
            pl.BlockSpec((Cout, 2 * Cin), lambda b, i: (0, 0)),
        ],
        out_specs=[
            pl.BlockSpec((1, R, Cout), lambda b, i: (b, i, 0)),
            pl.BlockSpec((1, 2, Cout), lambda b, i: (b * NB + i, 0, 0)),
        ],
        out_shape=[
            jax.ShapeDtypeStruct((B, N, Cout), jnp.float32),
            jax.ShapeDtypeStruct((B * NB, 2, Cout), jnp.float32),
        ],
    )


@functools.lru_cache(maxsize=None)
def _build_conv_y(B, N, R, Cin, Cout):
    NB = N // R

    def kern(xg_ref, x_ref, w_ref, m_ref, y_ref):
        xg = xg_ref[0][:, :Cin]
        ctr = x_ref[0][:, :Cin]
        ctr_rep = jnp.broadcast_to(
            ctr[:, None, :], (R, KNN, Cin)).reshape(R * KNN, Cin)
        feat = jnp.concatenate([xg - ctr_rep, ctr_rep], axis=1)
        w = w_ref[...]
        y = _dot(feat, w)
        m_ref[0] = jnp.max(y.reshape(R, KNN, Cout), axis=1)
        yt = lax.dot_general(w, feat, (((1,), (1,)), ((), ())),
                             preferred_element_type=jnp.float32)
        y_ref[0] = yt.reshape(Cout, R, KNN)

    return pl.pallas_call(
        kern,
        grid=(B, NB),
        in_specs=[
            pl.BlockSpec((1, R * KNN, CP), lambda b, i: (b * NB + i, 0, 0)),
            pl.BlockSpec((1, R, CP), lambda b, i: (b, i, 0)),
            pl.BlockSpec((Cout, 2 * Cin), lambda b, i: (0, 0)),
        ],
        out_specs=[
            pl.BlockSpec((1, R, Cout), lambda b, i: (b, i, 0)),
            pl.BlockSpec((1, Cout, R, KNN), lambda b, i: (b, 0, i, 0)),
        ],
        out_shape=[
            jax.ShapeDtypeStruct((B, N, Cout), jnp.float32),
            jax.ShapeDtypeStruct((B, Cout, N, KNN), jnp.float32),
        ],
    )


@functools.lru_cache(maxsize=None)
def _build_bn_act2(B, N, C, Cpad):
    def kern(m_ref, mean_ref, var_ref, g_ref, b_ref, o_ref):
        sd = jnp.sqrt(var_ref[...] + 1e-5)
        y = (m_ref[0] - mean_ref[...]) / sd * g_ref[...] + b_ref[...]
        y = jnp.where(y > 0, y, 0.2 * y)
        if Cpad > C:
            y = jnp.concatenate(
                [y, jnp.zeros((N, Cpad - C), jnp.float32)], axis=1)
        o_ref[0] = y

    return pl.pallas_call(
        kern,
        grid=(B,),
        in_specs=[
            pl.BlockSpec((1, N, C), lambda b: (b, 0, 0)),
            pl.BlockSpec((1, C), lambda b: (0, 0)),
            pl.BlockSpec((1, C), lambda b: (0, 0)),
            pl.BlockSpec((1, C), lambda b: (0, 0)),
            pl.BlockSpec((1, C), lambda b: (0, 0)),
        ],
        out_specs=pl.BlockSpec((1, N, Cpad), lambda b: (b, 0, 0)),
        out_shape=jax.ShapeDtypeStruct((B, N, Cpad), jnp.float32),
    )


@functools.lru_cache(maxsize=None)
def _build_conv_var(B, N, R, Cin, Cout, count):
    NB = N // R
    inv = 1.0 / float(count)

    def kern(xg_ref, x_ref, w_ref, s_ref, q_ref):
        xg = xg_ref[0][:, :Cin]
        ctr = x_ref[0][:, :Cin]
        ctr_rep = jnp.broadcast_to(
            ctr[:, None, :], (R, KNN, Cin)).reshape(R * KNN, Cin)
        feat = jnp.concatenate([xg - ctr_rep, ctr_rep], axis=1)
        y = _dot(feat, w_ref[...])
        s = s_ref[...]
        mean = _comp_combine(s[:, 0, :], s[:, 1, :]) * inv
        d = y - mean
        q_hi, q_lo = _comp_colsum(d * d)
        q_ref[0] = jnp.concatenate([q_hi, q_lo], axis=0)

    return pl.pallas_call(
        kern,
        grid=(B, NB),
        in_specs=[
            pl.BlockSpec((1, R * KNN, CP), lambda b, i: (b * NB + i, 0, 0)),
            pl.BlockSpec((1, R, CP), lambda b, i: (b, i, 0)),
            pl.BlockSpec((Cout, 2 * Cin), lambda b, i: (0, 0)),
            pl.BlockSpec((B * NB, 2, Cout), lambda b, i: (0, 0, 0)),
        ],
        out_specs=pl.BlockSpec((1, 2, Cout), lambda b, i: (b * NB + i, 0, 0)),
        out_shape=jax.ShapeDtypeStruct((B * NB, 2, Cout), jnp.float32),
    )


@functools.lru_cache(maxsize=None)
def _build_bn_act(B, N, C, NP, count, Cpad):
    inv = 1.0 / float(count)

    def kern(m_ref, s_ref, q_ref, g_ref, b_ref, o_ref):
        s = s_ref[...]
        q = q_ref[...]
        mean = _comp_combine(s[:, 0, :], s[:, 1, :]) * inv
        var = _comp_combine(q[:, 0, :], q[:, 1, :]) * inv
        sd = jnp.sqrt(var + 1e-5)
        y = (m_ref[0] - mean) / sd * g_ref[...] + b_ref[...]
        y = jnp.where(y > 0, y, 0.2 * y)
        if Cpad > C:
            y = jnp.concatenate(
                [y, jnp.zeros((N, Cpad - C), jnp.float32)], axis=1)
        o_ref[0] = y

    return pl.pallas_call(
        kern,
        grid=(B,),
        in_specs=[
            pl.BlockSpec((1, N, C), lambda b: (b, 0, 0)),
            pl.BlockSpec((NP, 2, C), lambda b: (0, 0, 0)),
            pl.BlockSpec((NP, 2, C), lambda b: (0, 0, 0)),
            pl.BlockSpec((1, C), lambda b: (0, 0)),
            pl.BlockSpec((1, C), lambda b: (0, 0)),
        ],
        out_specs=pl.BlockSpec((1, N, Cpad), lambda b: (b, 0, 0)),
        out_shape=jax.ShapeDtypeStruct((B, N, Cpad), jnp.float32),
    )


@functools.lru_cache(maxsize=None)
def _build_conv5(B, N, Co):
    def kern(x1, x2, x3, x4, w, ymax, ys, yq):
        cat = jnp.concatenate(
            [x1[0][:, :64], x2[0][:, :64], x3[0], x4[0]], axis=1)
        y = _dot(cat, w[...])
        ymax[0] = jnp.max(y, axis=0, keepdims=True)
        s = jnp.sum(y, axis=0, keepdims=True)
        ys[0] = s
        d = y - s * (1.0 / float(N))
        yq[0] = jnp.sum(d * d, axis=0, keepdims=True)

    return pl.pallas_call(
        kern,
        grid=(B,),
        in_specs=[
            pl.BlockSpec((1, N, CP), lambda b: (b, 0, 0)),
            pl.BlockSpec((1, N, CP), lambda b: (b, 0, 0)),
            pl.BlockSpec((1, N, 128), lambda b: (b, 0, 0)),
            pl.BlockSpec((1, N, 256), lambda b: (b, 0, 0)),
            pl.BlockSpec((Co, 512), lambda b: (0, 0)),
        ],
        out_specs=[
            pl.BlockSpec((1, 1, Co), lambda b: (b, 0, 0)),
            pl.BlockSpec((1, 1, Co), lambda b: (b, 0, 0)),
            pl.BlockSpec((1, 1, Co), lambda b: (b, 0, 0)),
        ],
        out_shape=[
            jax.ShapeDtypeStruct((B, 1, Co), jnp.float32),
            jax.ShapeDtypeStruct((B, 1, Co), jnp.float32),
            jax.ShapeDtypeStruct((B, 1, Co), jnp.float32),
        ],
    )


@functools.lru_cache(maxsize=None)
def _build_heads(B, N):
    inv = 1.0 / float(B * N)

    def kern(ymax, ys, yq, g, bb, wmu, bmu, wlv, blv, wd1, bd1,
             wd2, bd2, wp, bp, eps, rec_o, mu_o, lv_o, proj_o):
        s_rows = ys[...]
        s = jnp.sum(s_rows, axis=0, keepdims=True)
        q = jnp.sum(yq[...], axis=0, keepdims=True)
        mean = s * inv
        d = s_rows * (1.0 / float(N)) - mean
        var = (q + float(N) * jnp.sum(d * d, axis=0, keepdims=True)) * inv
        sd = jnp.sqrt(var + 1e-5)
        t = (ymax[...] - mean) / sd * g[...] + bb[...]
        feat = jnp.where(t > 0, t, 0.2 * t)
        mu = _dot(feat, wmu[...]) + bmu[...]
        lv = _dot(feat, wlv[...]) + blv[...]
        z = mu + eps[...] * jnp.exp(0.5 * lv)
        h = jnp.maximum(_dot(z, wd1[...]) + bd1[...], 0.0)
        rec_o[...] = jnp.tanh(_dot(h, wd2[...]) + bd2[...])
        mu_o[...] = mu
        lv_o[...] = lv
        proj_o[...] = _dot(feat, wp[...]) + bp[...]

    return pl.pallas_call(
        kern,
        out_shape=[
            jax.ShapeDtypeStruct((B, 3072), jnp.float32),
            jax.ShapeDtypeStruct((B, 128), jnp.float32),
            jax.ShapeDtypeStruct((B, 128), jnp.float32),
            jax.ShapeDtypeStruct((B, 128), jnp.float32),
        ],
    )


def kernel(x, W1, g1, b1, W2, g2, b2, W3, g3, b3, W4, g4, b4, W5, g5, b5,
           Wmu, bmu, Wlv, blv, Wd1, bd1, Wd2, bd2, Wp, bp, eps):
    B, Cx, N = x.shape
    R = 128
    X = jnp.pad(jnp.transpose(x, (0, 2, 1)), ((0, 0), (0, 0), (0, CP - Cx)))
    layers = [
        (W1, Cx, g1, b1),
        (W2, 64, g2, b2),
        (W3, 64, g3, b3),
        (W4, 128, g4, b4),
    ]
    feats = []
    for li, (Wf, Cin, g, bb) in enumerate(layers):
        Cout = Wf.shape[0]
        idx = _build_edge(B, N, 256)(X, X)
        xg = _build_gather(B * N)(X.reshape(B * N, CP),
                                  idx.reshape(B * N * KNN))
        xg3 = xg.reshape(B * (N // R), R * KNN, CP)
        if li < 3:
            M, Y = _build_conv_y(B, N, R, Cin, Cout)(xg3, X, Wf)
            Y = lax.optimization_barrier(Y)
            mean = jnp.mean(Y, axis=(0, 2, 3)).reshape(1, Cout)
            var = jnp.var(Y, axis=(0, 2, 3)).reshape(1, Cout)
            X = _build_bn_act2(B, N, Cout, CP)(
                M, mean, var, g.reshape(1, Cout), bb.reshape(1, Cout))
        else:
            M, S = _build_conv(B, N, R, Cin, Cout)(xg3, X, Wf)
            Q = _build_conv_var(B, N, R, Cin, Cout, B * N * KNN)(xg3, X, Wf, S)
            X = _build_bn_act(B, N, Cout, B * (N // R), B * N * KNN,
                              CP if li < 3 else Cout)(
                M, S, Q, g.reshape(1, Cout), bb.reshape(1, Cout))
        feats.append(X)
    ymax, ys, yq = _build_conv5(B, N, 1024)(
        feats[0], feats[1], feats[2], feats[3], W5)
    ymax, ys, yq = (ymax.reshape(B, -1), ys.reshape(B, -1), yq.reshape(B, -1))
    rec, mu, lv, proj = _build_heads(B, N)(
        ymax, ys, yq, g5.reshape(1, -1), b5.reshape(1, -1),
        Wmu, bmu.reshape(1, -1), Wlv, blv.reshape(1, -1),
        Wd1, bd1.reshape(1, -1), Wd2, bd2.reshape(1, -1),
        Wp, bp.reshape(1, -1), eps)
    return rec.reshape(B, 3, N), mu, lv, proj
